# baseline (speedup 1.0000x reference)
import functools

import jax
import jax.numpy as jnp
from jax import lax
from jax.experimental import pallas as pl
from jax.experimental.pallas import tpu as pltpu
from jax.experimental.pallas import tpu_sc as plsc

NUM_WORKERS = 32
BBLK = 128
BPAD = BBLK + 1
LANES = 16
UPAIR = 2


def _make_kernel(b: int, l: int, embed: int):
    n_units = l * (b // BBLK)
    per_w = n_units // NUM_WORKERS
    n_pairs = per_w // UPAIR
    n_groups = n_pairs // 2
    eblk = embed // 8
    bb_per_l = b // BBLK

    mesh = plsc.VectorSubcoreMesh(core_axis_name="c", subcore_axis_name="s")

    @functools.partial(
        pl.kernel,
        mesh=mesh,
        out_type=jax.ShapeDtypeStruct((l * eblk, bb_per_l, 8, BBLK), jnp.float32),
        scratch_types=[
            pltpu.VMEM((n_pairs, UPAIR * BBLK), jnp.int32),
            pltpu.VMEM((2, UPAIR * BBLK, embed), jnp.float32),
            pltpu.VMEM((2, UPAIR, eblk, 8, BPAD), jnp.float32),
            pltpu.SemaphoreType.DMA((2,)),
            pltpu.SemaphoreType.DMA((2,)),
        ],
        compiler_params=pltpu.CompilerParams(
            use_tc_tiling_on_sc=False, needs_layout_passes=False
        ),
    )
    def gather_kernel(qlin_hbm, table_hbm, out_hbm, idx_v, rows_v, buf_v,
                      gsem, wsem):
        wid = lax.axis_index("s") * 2 + lax.axis_index("c")
        p0 = wid * n_pairs
        u0 = wid * per_w

        pltpu.sync_copy(qlin_hbm.at[pl.ds(p0, n_pairs)], idx_v)

        def gather_start(slot, g):
            pltpu.async_copy(
                table_hbm.at[idx_v.at[g]],
                rows_v.at[slot],
                gsem.at[slot],
            )

        def gather_wait(slot):
            pltpu.make_async_copy(
                table_hbm.at[idx_v.at[0]],
                rows_v.at[slot],
                gsem.at[slot],
            ).wait()

        def write_wait(slot):
            for h in range(UPAIR):
                pltpu.make_async_copy(
                    buf_v.at[slot, h, :, :, pl.ds(0, BBLK)],
                    out_hbm.at[pl.ds(0, eblk), 0],
                    wsem.at[slot],
                ).wait()

        e_vecs = [
            (
                (lax.iota(jnp.int32, LANES) + k * LANES) // 8,
                (lax.iota(jnp.int32, LANES) + k * LANES) % 8,
            )
            for k in range(embed // LANES)
        ]

        def transpose_unit(slot, h):
            for bc0 in range(0, BBLK, 8):
                for k in range(embed // LANES):
                    vals = [
                        rows_v[slot, h * BBLK + bc0 + j, pl.ds(k * LANES, LANES)]
                        for j in range(8)
                    ]
                    for j in range(8):
                        plsc.store_scatter(
                            buf_v.at[slot, h],
                            [
                                e_vecs[k][0],
                                e_vecs[k][1],
                                jnp.full((LANES,), bc0 + j, jnp.int32),
                            ],
                            vals[j],
                        )

        def write_start(slot, h, u):
            l_id = u // bb_per_l
            bb = u - l_id * bb_per_l
            pltpu.async_copy(
                buf_v.at[slot, h, :, :, pl.ds(0, BBLK)],
                out_hbm.at[pl.ds(l_id * eblk, eblk), bb],
                wsem.at[slot],
            )

        gather_start(0, 0)
        gather_start(1, 1)

        def body(g2, carry):
            g0 = g2 * 2
            for slot in range(2):
                g = g0 + slot
                gather_wait(slot)

                @pl.when(g2 > 0)
                def _():
                    write_wait(slot)

                for h in range(UPAIR):
                    transpose_unit(slot, h)
                    write_start(slot, h, u0 + g * UPAIR + h)

                @pl.when(g2 + 1 < n_groups)
                def _():
                    gather_start(slot, g + 2)

            return carry

        lax.fori_loop(0, n_groups, body, 0)
        write_wait(0)
        write_wait(1)

    return gather_kernel


def kernel(q, W):
    b, l = q.shape
    _, embed = W.shape
    qlin = q.T.reshape(l * (b // BBLK) // UPAIR, UPAIR * BBLK).astype(jnp.int32)
    out = _make_kernel(b, l, embed)(qlin, W)
    x5 = out.reshape(l, embed // 8, b // BBLK, 8, BBLK)
    return x5.transpose(2, 4, 0, 1, 3).reshape(b, l, embed)
